# baseline (speedup 1.0000x reference)
import functools

import jax
import jax.numpy as jnp
from jax import lax
from jax.experimental import pallas as pl
from jax.experimental.pallas import tpu as pltpu
from jax.experimental.pallas import tpu_sc as plsc

DAY_SIZE = 32
TIME_SIZE = 32

BS = 16
ROWS_PER_W = 48
PAD_W = 128


def _make_sc_gather(n_rows):
    n_workers = n_rows // ROWS_PER_W
    mesh = plsc.VectorSubcoreMesh(core_axis_name="c", subcore_axis_name="s")

    @functools.partial(
        pl.kernel,
        mesh=mesh,
        out_type=jax.ShapeDtypeStruct((n_rows, PAD_W), jnp.float32),
        scratch_types=[
            pltpu.VMEM((ROWS_PER_W,), jnp.int32),
            pltpu.VMEM((ROWS_PER_W, PAD_W), jnp.float32),
            pltpu.SemaphoreType.DMA,
        ],
    )
    def sc_gather(table_hbm, idx_hbm, rows_hbm, idx_v, rows_v, sem):
        wid = lax.axis_index("s")
        cid = lax.axis_index("c")

        @pl.when((cid == 0) & (wid < n_workers))
        def _():
            base = wid * ROWS_PER_W
            pltpu.sync_copy(idx_hbm.at[pl.ds(base, ROWS_PER_W)], idx_v)
            pltpu.async_copy(table_hbm.at[idx_v], rows_v, sem).wait()
            pltpu.sync_copy(rows_v, rows_hbm.at[pl.ds(base, ROWS_PER_W)])

    return sc_gather


def _embed_block_kernel(eday_ref, etime_ref, wnode_ref, out_ref):
    node_count = out_ref.shape[1]
    node_part = wnode_ref[...]
    for j in range(BS):
        block = jnp.concatenate(
            (
                jnp.broadcast_to(
                    eday_ref[pl.ds(j, 1), 0:DAY_SIZE], (node_count, DAY_SIZE)
                ),
                jnp.broadcast_to(
                    etime_ref[pl.ds(j, 1), 0:TIME_SIZE], (node_count, TIME_SIZE)
                ),
                node_part,
            ),
            axis=-1,
        )
        out_ref[j] = block


def kernel(daytime, W_day, W_time, W_node):
    batch, seq, _ = daytime.shape
    node_count, node_size = W_node.shape
    bs = batch * seq
    feat = DAY_SIZE + TIME_SIZE + node_size
    day_count = W_day.shape[0]

    table = jnp.concatenate(
        (
            jnp.pad(W_day, ((0, 0), (0, PAD_W - DAY_SIZE))),
            jnp.pad(W_time, ((0, 0), (0, PAD_W - TIME_SIZE))),
        ),
        axis=0,
    )
    d_idx = daytime[..., 0].reshape(bs)
    t_idx = daytime[..., 1].reshape(bs) + day_count
    idx = jnp.concatenate((d_idx, t_idx))

    rows = _make_sc_gather(2 * bs)(table, idx)

    n_blk = bs // BS
    out = pl.pallas_call(
        _embed_block_kernel,
        grid=(n_blk,),
        in_specs=[
            pl.BlockSpec((BS, PAD_W), lambda i: (i, 0)),
            pl.BlockSpec((BS, PAD_W), lambda i: (i + n_blk, 0)),
            pl.BlockSpec(W_node.shape, lambda i: (0, 0)),
        ],
        out_specs=pl.BlockSpec((BS, node_count, feat), lambda i: (i, 0, 0)),
        out_shape=jax.ShapeDtypeStruct((bs, node_count, feat), jnp.float32),
    )(rows, rows, W_node)
    return out.reshape(batch, seq, node_count, feat)


# speedup vs baseline: 1.0046x; 1.0046x over previous
import functools

import jax
import jax.numpy as jnp
from jax import lax
from jax.experimental import pallas as pl
from jax.experimental.pallas import tpu as pltpu
from jax.experimental.pallas import tpu_sc as plsc

DAY_SIZE = 32
TIME_SIZE = 32

BS = 8
ROWS_PER_W = 48
PAD_W = 128


def _make_sc_gather(n_rows):
    n_workers = n_rows // ROWS_PER_W
    mesh = plsc.VectorSubcoreMesh(core_axis_name="c", subcore_axis_name="s")

    @functools.partial(
        pl.kernel,
        mesh=mesh,
        out_type=jax.ShapeDtypeStruct((n_rows, PAD_W), jnp.float32),
        scratch_types=[
            pltpu.VMEM((ROWS_PER_W,), jnp.int32),
            pltpu.VMEM((ROWS_PER_W, PAD_W), jnp.float32),
            pltpu.SemaphoreType.DMA,
        ],
    )
    def sc_gather(table_hbm, idx_hbm, rows_hbm, idx_v, rows_v, sem):
        wid = lax.axis_index("s")
        cid = lax.axis_index("c")

        @pl.when((cid == 0) & (wid < n_workers))
        def _():
            base = wid * ROWS_PER_W
            pltpu.sync_copy(idx_hbm.at[pl.ds(base, ROWS_PER_W)], idx_v)
            pltpu.async_copy(table_hbm.at[idx_v], rows_v, sem).wait()
            pltpu.sync_copy(rows_v, rows_hbm.at[pl.ds(base, ROWS_PER_W)])

    return sc_gather


def _embed_block_kernel(eday_ref, etime_ref, wnode_ref, out_ref):
    node_count = out_ref.shape[1]
    node_part = wnode_ref[...]
    for j in range(BS):
        block = jnp.concatenate(
            (
                jnp.broadcast_to(
                    eday_ref[pl.ds(j, 1), 0:DAY_SIZE], (node_count, DAY_SIZE)
                ),
                jnp.broadcast_to(
                    etime_ref[pl.ds(j, 1), 0:TIME_SIZE], (node_count, TIME_SIZE)
                ),
                node_part,
            ),
            axis=-1,
        )
        out_ref[j] = block


def kernel(daytime, W_day, W_time, W_node):
    batch, seq, _ = daytime.shape
    node_count, node_size = W_node.shape
    bs = batch * seq
    feat = DAY_SIZE + TIME_SIZE + node_size
    day_count = W_day.shape[0]

    table = jnp.concatenate(
        (
            jnp.pad(W_day, ((0, 0), (0, PAD_W - DAY_SIZE))),
            jnp.pad(W_time, ((0, 0), (0, PAD_W - TIME_SIZE))),
        ),
        axis=0,
    )
    d_idx = daytime[..., 0].reshape(bs)
    t_idx = daytime[..., 1].reshape(bs) + day_count
    idx = jnp.concatenate((d_idx, t_idx))

    rows = _make_sc_gather(2 * bs)(table, idx)

    n_blk = bs // BS
    out = pl.pallas_call(
        _embed_block_kernel,
        grid=(n_blk,),
        in_specs=[
            pl.BlockSpec((BS, PAD_W), lambda i: (i, 0)),
            pl.BlockSpec((BS, PAD_W), lambda i: (i + n_blk, 0)),
            pl.BlockSpec(W_node.shape, lambda i: (0, 0)),
        ],
        out_specs=pl.BlockSpec((BS, node_count, feat), lambda i: (i, 0, 0)),
        out_shape=jax.ShapeDtypeStruct((bs, node_count, feat), jnp.float32),
    )(rows, rows, W_node)
    return out.reshape(batch, seq, node_count, feat)


# speedup vs baseline: 1.2213x; 1.2157x over previous
import jax
import jax.numpy as jnp
from jax.experimental import pallas as pl
from jax.experimental.pallas import tpu as pltpu

DAY_SIZE = 32
TIME_SIZE = 32
SLOTS = 16


def _embed_ring_kernel(idx_ref, wday_ref, wtime_ref, wnode_ref, out_hbm,
                       scratch, sems):
    g = pl.program_id(0)
    n_steps = pl.num_programs(0)
    node_count = out_hbm.shape[1]
    node_part = wnode_ref[...]
    for j in range(SLOTS):
        p = g * SLOTS + j

        @pl.when(g > 0)
        def _():
            pltpu.make_async_copy(
                scratch.at[pl.ds(j, 1)], out_hbm.at[pl.ds(p - SLOTS, 1)], sems.at[j]
            ).wait()

        d = idx_ref[p, 0]
        t = idx_ref[p, 1]
        block = jnp.concatenate(
            (
                jnp.broadcast_to(wday_ref[pl.ds(d, 1), :], (node_count, DAY_SIZE)),
                jnp.broadcast_to(wtime_ref[pl.ds(t, 1), :], (node_count, TIME_SIZE)),
                node_part,
            ),
            axis=-1,
        )
        scratch[j] = block
        pltpu.make_async_copy(
            scratch.at[pl.ds(j, 1)], out_hbm.at[pl.ds(p, 1)], sems.at[j]
        ).start()

    @pl.when(g == n_steps - 1)
    def _():
        for j in range(SLOTS):
            pltpu.make_async_copy(
                scratch.at[pl.ds(j, 1)], out_hbm.at[pl.ds(g * SLOTS + j, 1)], sems.at[j]
            ).wait()


def kernel(daytime, W_day, W_time, W_node):
    batch, seq, _ = daytime.shape
    node_count, node_size = W_node.shape
    bs = batch * seq
    feat = DAY_SIZE + TIME_SIZE + node_size
    idx = daytime.reshape(bs, 2)

    grid_spec = pltpu.PrefetchScalarGridSpec(
        num_scalar_prefetch=1,
        grid=(bs // SLOTS,),
        in_specs=[
            pl.BlockSpec(W_day.shape, lambda i, idx_ref: (0, 0)),
            pl.BlockSpec(W_time.shape, lambda i, idx_ref: (0, 0)),
            pl.BlockSpec(W_node.shape, lambda i, idx_ref: (0, 0)),
        ],
        out_specs=pl.BlockSpec(memory_space=pl.MemorySpace.ANY),
        scratch_shapes=[
            pltpu.VMEM((SLOTS, node_count, feat), jnp.float32),
            pltpu.SemaphoreType.DMA((SLOTS,)),
        ],
    )
    out = pl.pallas_call(
        _embed_ring_kernel,
        grid_spec=grid_spec,
        out_shape=jax.ShapeDtypeStruct((bs, node_count, feat), jnp.float32),
    )(idx, W_day, W_time, W_node)
    return out.reshape(batch, seq, node_count, feat)
